# baseline (speedup 1.0000x reference)
import functools

import jax
import jax.numpy as jnp
from jax import lax
from jax.experimental import pallas as pl
from jax.experimental.pallas import tpu as pltpu
from jax.experimental.pallas import tpu_sc as plsc

BATCH = 16384
DIM = 128

NUM_CORES = 2
NUM_SUBCORES = 16
NUM_WORKERS = NUM_CORES * NUM_SUBCORES
BPW = BATCH // NUM_WORKERS
CHUNK = 128
CHUNKS_PER_OUT = BPW // CHUNK
NBUF = 4


def _gather_body(re_hbm, im_hbm, i_hbm, out_re, out_im,
                 idx, bufs, gsems, wsems):
    wid = lax.axis_index("s") * NUM_CORES + lax.axis_index("c")
    base = wid * BPW

    pltpu.sync_copy(i_hbm.at[pl.ds(base, BPW)], idx)

    tasks = []
    for table, out in ((re_hbm, out_re), (im_hbm, out_im)):
        for c in range(CHUNKS_PER_OUT):
            tasks.append((table, out, c))

    def start_gather(t):
        table, _, c = tasks[t]
        b = t % NBUF
        pltpu.async_copy(table.at[idx.at[pl.ds(c * CHUNK, CHUNK)]],
                         bufs[b], gsems[b])

    def wait_gather(b):
        pltpu.make_async_copy(re_hbm.at[pl.ds(0, CHUNK)], bufs[b],
                              gsems[b]).wait()

    def wait_write(b, out):
        pltpu.make_async_copy(bufs[b], out.at[pl.ds(base, CHUNK)],
                              wsems[b]).wait()

    for t in range(NBUF):
        start_gather(t)

    for t in range(len(tasks)):
        b = t % NBUF
        _, out, c = tasks[t]
        wait_gather(b)
        row0 = base + c * CHUNK
        pltpu.async_copy(bufs[b], out.at[pl.ds(row0, CHUNK)], wsems[b])
        if t + NBUF < len(tasks):
            wait_write(b, out)
            start_gather(t + NBUF)

    for t in range(len(tasks) - NBUF, len(tasks)):
        b = t % NBUF
        wait_write(b, tasks[t][1])


_sc_gather = functools.partial(
    pl.kernel,
    out_type=[jax.ShapeDtypeStruct((BATCH, DIM), jnp.float32)] * 2,
    mesh=plsc.VectorSubcoreMesh(core_axis_name="c", subcore_axis_name="s"),
    scratch_types=(
        [pltpu.VMEM((BPW,), jnp.int32)]
        + [[pltpu.VMEM((CHUNK, DIM), jnp.float32) for _ in range(NBUF)]]
        + [[pltpu.SemaphoreType.DMA for _ in range(NBUF)]]
        + [[pltpu.SemaphoreType.DMA for _ in range(NBUF)]]
    ),
)


def kernel(inputs, entity_embedding_real, entity_embedding_img,
           relation_embedding_real, relation_embedding_img):
    s = inputs[:, 0].astype(jnp.int32)
    p = inputs[:, 1].astype(jnp.int32)
    o = inputs[:, 2].astype(jnp.int32)

    g = _sc_gather(_gather_body)
    sr, si = g(entity_embedding_real, entity_embedding_img, s)
    pr, pi = g(relation_embedding_real, relation_embedding_img, p)
    orr, oi = g(entity_embedding_real, entity_embedding_img, o)
    return (lax.complex(sr, si), lax.complex(pr, pi), lax.complex(orr, oi))
